# baseline (speedup 1.0000x reference)
import functools

import jax
import jax.numpy as jnp
from jax import lax
from jax.experimental import pallas as pl
from jax.experimental.pallas import tpu as pltpu
from jax.experimental.pallas import tpu_sc as plsc

_NC, _NS = 2, 16
_NW = _NC * _NS
_TCW = 16


@functools.lru_cache(maxsize=None)
def _make_sc_split(R, Rt, H, W):
    bpw = Rt // _NW
    tab_w = (bpw + 15) // 16 * 16
    assert Rt % _NW == 0 and bpw <= 32

    mesh = plsc.VectorSubcoreMesh(
        core_axis_name="c", subcore_axis_name="s",
        num_cores=_NC, num_subcores=_NS,
    )

    @functools.partial(
        pl.kernel,
        out_type=jax.ShapeDtypeStruct((Rt, H, W), jnp.float32),
        mesh=mesh,
        scratch_types=[
            pltpu.VMEM((tab_w,), jnp.int32),
            pltpu.VMEM((H, W), jnp.float32),
            pltpu.VMEM((H, W), jnp.float32),
            pltpu.SemaphoreType.DMA,
            pltpu.SemaphoreType.DMA,
            pltpu.SemaphoreType.DMA,
            pltpu.SemaphoreType.DMA,
        ],
    )
    def sc_kernel(img, tab, out, tab_v, buf0, buf1, sg0, sg1, ss0, ss1):
        w = lax.axis_index("s") * _NC + lax.axis_index("c")
        pltpu.sync_copy(tab.at[w], tab_v)
        svs = [tab_v[pl.ds(16 * i, 16)] for i in range(tab_w // 16)]
        bufs, gsems, ssems = (buf0, buf1), (sg0, sg1), (ss0, ss1)

        def gat(k):
            sv, lane = svs[k // 16], k % 16
            src = lax.squeeze(lax.slice(sv, (lane,), (lane + 1,)), (0,))
            return pltpu.make_async_copy(img.at[src], bufs[k % 2], gsems[k % 2])

        def sca(k):
            return pltpu.make_async_copy(
                bufs[k % 2], out.at[w * bpw + k], ssems[k % 2])

        gat(0).start()
        for k in range(bpw):
            if k + 1 < bpw:
                if k >= 1:
                    sca(k - 1).wait()
                gat(k + 1).start()
            gat(k).wait()
            sca(k).start()
        for j in range(max(0, bpw - 2), bpw):
            sca(j).wait()

    return sc_kernel


@functools.lru_cache(maxsize=None)
def _make_tc_split(R, Ri, H, W):
    def tc_kernel(idx_s, img, out, sem):
        def cp(k):
            return pltpu.make_async_copy(img.at[idx_s[k]], out.at[k], sem)

        def body(k, carry):
            @pl.when(k >= _TCW)
            def _():
                cp(k - _TCW).wait()
            cp(k).start()
            return carry

        lax.fori_loop(0, Ri, body, 0)

        def drain(k, carry):
            cp(k).wait()
            return carry

        lax.fori_loop(max(Ri - _TCW, 0), Ri, drain, 0)

    return pl.pallas_call(
        tc_kernel,
        grid_spec=pltpu.PrefetchScalarGridSpec(
            num_scalar_prefetch=1,
            in_specs=[pl.BlockSpec(memory_space=pl.ANY)],
            out_specs=pl.BlockSpec(memory_space=pl.ANY),
            scratch_shapes=[pltpu.SemaphoreType.DMA],
        ),
        out_shape=jax.ShapeDtypeStruct((Ri, H, W), jnp.float32),
    )


def kernel(image, context_indices, target_indices):
    B, V, C, H, W = image.shape
    ni = context_indices.shape[1]
    nt = target_indices.shape[1]
    Ri, Rt = B * ni * C, B * nt * C

    img3 = image.reshape(B * V * C, H, W)
    bi = jnp.arange(B, dtype=jnp.int32)[:, None]
    ch = jnp.arange(C, dtype=jnp.int32)[None, None, :]
    src_i = (((bi * V + context_indices) * C)[..., None] + ch).reshape(-1)
    src_t = (((bi * V + target_indices) * C)[..., None] + ch).reshape(_NW, -1)
    bpw_t = Rt // _NW
    pad = (bpw_t + 15) // 16 * 16 - bpw_t
    tab_t = jnp.concatenate([src_t, jnp.zeros((_NW, pad), jnp.int32)], axis=1)

    out_t = _make_sc_split(B * V * C, Rt, H, W)(img3, tab_t)
    out_i = _make_tc_split(B * V * C, Ri, H, W)(src_i, img3)
    return (out_i.reshape(B, ni, C, H, W),
            out_t.reshape(B, nt, C, H, W),
            context_indices, target_indices)


# speedup vs baseline: 10.1484x; 10.1484x over previous
import functools

import jax
import jax.numpy as jnp
from jax import lax
from jax.experimental import pallas as pl
from jax.experimental.pallas import tpu as pltpu
from jax.experimental.pallas import tpu_sc as plsc

_NC, _NS = 2, 16
_NW = _NC * _NS
_TCW = 16


@functools.lru_cache(maxsize=None)
def _make_sc_split(R, Rt, H, W):
    bpw = Rt // _NW
    tab_w = (bpw + 15) // 16 * 16
    assert Rt % _NW == 0 and bpw <= 32

    mesh = plsc.VectorSubcoreMesh(
        core_axis_name="c", subcore_axis_name="s",
        num_cores=_NC, num_subcores=_NS,
    )

    @functools.partial(
        pl.kernel,
        out_type=jax.ShapeDtypeStruct((Rt, H, W), jnp.float32),
        mesh=mesh,
        scratch_types=[
            pltpu.VMEM((tab_w,), jnp.int32),
            pltpu.VMEM((H, W), jnp.float32),
            pltpu.VMEM((H, W), jnp.float32),
            pltpu.SemaphoreType.DMA,
            pltpu.SemaphoreType.DMA,
            pltpu.SemaphoreType.DMA,
            pltpu.SemaphoreType.DMA,
        ],
    )
    def sc_kernel(img, tab, out, tab_v, buf0, buf1, sg0, sg1, ss0, ss1):
        w = lax.axis_index("s") * _NC + lax.axis_index("c")
        pltpu.sync_copy(tab.at[w], tab_v)
        svs = [tab_v[pl.ds(16 * i, 16)] for i in range(tab_w // 16)]
        bufs, gsems, ssems = (buf0, buf1), (sg0, sg1), (ss0, ss1)

        def gat(k):
            sv, lane = svs[k // 16], k % 16
            src = lax.squeeze(lax.slice(sv, (lane,), (lane + 1,)), (0,))
            return pltpu.make_async_copy(img.at[src], bufs[k % 2], gsems[k % 2])

        def sca(k):
            return pltpu.make_async_copy(
                bufs[k % 2], out.at[w * bpw + k], ssems[k % 2])

        gat(0).start()
        for k in range(bpw):
            if k + 1 < bpw:
                if k >= 1:
                    sca(k - 1).wait()
                gat(k + 1).start()
            gat(k).wait()
            sca(k).start()
        for j in range(max(0, bpw - 2), bpw):
            sca(j).wait()

    return sc_kernel


@functools.lru_cache(maxsize=None)
def _make_tc_split(R, Ri, H, W):
    def tc_kernel(idx_s, blk, out):
        out[...] = blk[...]

    return pl.pallas_call(
        tc_kernel,
        grid_spec=pltpu.PrefetchScalarGridSpec(
            num_scalar_prefetch=1,
            grid=(Ri,),
            in_specs=[pl.BlockSpec((1, H, W), lambda k, idx: (idx[k], 0, 0))],
            out_specs=pl.BlockSpec((1, H, W), lambda k, idx: (k, 0, 0)),
        ),
        out_shape=jax.ShapeDtypeStruct((Ri, H, W), jnp.float32),
    )


def kernel(image, context_indices, target_indices):
    B, V, C, H, W = image.shape
    ni = context_indices.shape[1]
    nt = target_indices.shape[1]
    Ri, Rt = B * ni * C, B * nt * C

    img3 = image.reshape(B * V * C, H, W)
    bi = jnp.arange(B, dtype=jnp.int32)[:, None]
    ch = jnp.arange(C, dtype=jnp.int32)[None, None, :]
    src_i = (((bi * V + context_indices) * C)[..., None] + ch).reshape(-1)
    src_t = (((bi * V + target_indices) * C)[..., None] + ch).reshape(_NW, -1)
    bpw_t = Rt // _NW
    pad = (bpw_t + 15) // 16 * 16 - bpw_t
    tab_t = jnp.concatenate([src_t, jnp.zeros((_NW, pad), jnp.int32)], axis=1)

    out_t = _make_sc_split(B * V * C, Rt, H, W)(img3, tab_t)
    out_i = _make_tc_split(B * V * C, Ri, H, W)(src_i, img3)
    return (out_i.reshape(B, ni, C, H, W),
            out_t.reshape(B, nt, C, H, W),
            context_indices, target_indices)


# speedup vs baseline: 18.8543x; 1.8579x over previous
import functools

import jax
import jax.numpy as jnp
from jax import lax
from jax.experimental import pallas as pl
from jax.experimental.pallas import tpu as pltpu
from jax.experimental.pallas import tpu_sc as plsc

_NC, _NS = 2, 16
_NW = _NC * _NS
_TCW = 16


@functools.lru_cache(maxsize=None)
def _make_sc_split(R, Rt, H, W):
    bpw = Rt // _NW
    tab_w = (bpw + 15) // 16 * 16
    assert Rt % _NW == 0 and bpw <= 32

    mesh = plsc.VectorSubcoreMesh(
        core_axis_name="c", subcore_axis_name="s",
        num_cores=_NC, num_subcores=_NS,
    )

    @functools.partial(
        pl.kernel,
        out_type=jax.ShapeDtypeStruct((Rt, H, W), jnp.float32),
        mesh=mesh,
        scratch_types=[
            pltpu.VMEM((tab_w,), jnp.int32),
            pltpu.VMEM((H, W), jnp.float32),
            pltpu.VMEM((H, W), jnp.float32),
            pltpu.SemaphoreType.DMA,
            pltpu.SemaphoreType.DMA,
            pltpu.SemaphoreType.DMA,
            pltpu.SemaphoreType.DMA,
        ],
    )
    def sc_kernel(img, tab, out, tab_v, buf0, buf1, sg0, sg1, ss0, ss1):
        w = lax.axis_index("s") * _NC + lax.axis_index("c")
        pltpu.sync_copy(tab.at[w], tab_v)
        svs = [tab_v[pl.ds(16 * i, 16)] for i in range(tab_w // 16)]
        bufs, gsems, ssems = (buf0, buf1), (sg0, sg1), (ss0, ss1)

        def gat(k):
            sv, lane = svs[k // 16], k % 16
            src = lax.squeeze(lax.slice(sv, (lane,), (lane + 1,)), (0,))
            return pltpu.make_async_copy(img.at[src], bufs[k % 2], gsems[k % 2])

        def sca(k):
            return pltpu.make_async_copy(
                bufs[k % 2], out.at[w * bpw + k], ssems[k % 2])

        gat(0).start()
        for k in range(bpw):
            if k + 1 < bpw:
                if k >= 1:
                    sca(k - 1).wait()
                gat(k + 1).start()
            gat(k).wait()
            sca(k).start()
        for j in range(max(0, bpw - 2), bpw):
            sca(j).wait()

    return sc_kernel


@functools.lru_cache(maxsize=None)
def _make_tc_split(R, Ni, C, H, W):
    def tc_kernel(idx_s, blk, out):
        out[...] = blk[...]

    return pl.pallas_call(
        tc_kernel,
        grid_spec=pltpu.PrefetchScalarGridSpec(
            num_scalar_prefetch=1,
            grid=(Ni,),
            in_specs=[pl.BlockSpec((1, C, H, W), lambda k, idx: (idx[k], 0, 0, 0))],
            out_specs=pl.BlockSpec((1, C, H, W), lambda k, idx: (k, 0, 0, 0)),
        ),
        out_shape=jax.ShapeDtypeStruct((Ni, C, H, W), jnp.float32),
    )


def kernel(image, context_indices, target_indices):
    B, V, C, H, W = image.shape
    ni = context_indices.shape[1]
    nt = target_indices.shape[1]
    Ri, Rt = B * ni * C, B * nt * C

    img3 = image.reshape(B * V * C, H, W)
    img4 = image.reshape(B * V, C, H, W)
    bi = jnp.arange(B, dtype=jnp.int32)[:, None]
    ch = jnp.arange(C, dtype=jnp.int32)[None, None, :]
    src_i = (bi * V + context_indices).reshape(-1)
    src_t = (((bi * V + target_indices) * C)[..., None] + ch).reshape(_NW, -1)
    bpw_t = Rt // _NW
    pad = (bpw_t + 15) // 16 * 16 - bpw_t
    tab_t = jnp.concatenate([src_t, jnp.zeros((_NW, pad), jnp.int32)], axis=1)

    out_t = _make_sc_split(B * V * C, Rt, H, W)(img3, tab_t)
    out_i = _make_tc_split(B * V, B * ni, C, H, W)(src_i, img4)
    return (out_i.reshape(B, ni, C, H, W),
            out_t.reshape(B, nt, C, H, W),
            context_indices, target_indices)


# speedup vs baseline: 25.1049x; 1.3315x over previous
import functools

import jax
import jax.numpy as jnp
from jax import lax
from jax.experimental import pallas as pl
from jax.experimental.pallas import tpu as pltpu
from jax.experimental.pallas import tpu_sc as plsc

_NC, _NS = 2, 16
_NW = _NC * _NS
_TCW = 16


@functools.lru_cache(maxsize=None)
def _make_sc_split(R, Rt, H, W):
    bpw = Rt // _NW
    tab_w = (bpw + 15) // 16 * 16
    assert Rt % _NW == 0 and bpw <= 32

    mesh = plsc.VectorSubcoreMesh(
        core_axis_name="c", subcore_axis_name="s",
        num_cores=_NC, num_subcores=_NS,
    )

    @functools.partial(
        pl.kernel,
        out_type=jax.ShapeDtypeStruct((Rt, H, W), jnp.float32),
        mesh=mesh,
        scratch_types=[
            pltpu.VMEM((tab_w,), jnp.int32),
            pltpu.VMEM((H, W), jnp.float32),
            pltpu.VMEM((H, W), jnp.float32),
            pltpu.SemaphoreType.DMA,
            pltpu.SemaphoreType.DMA,
            pltpu.SemaphoreType.DMA,
            pltpu.SemaphoreType.DMA,
        ],
    )
    def sc_kernel(img, tab, out, tab_v, buf0, buf1, sg0, sg1, ss0, ss1):
        w = lax.axis_index("s") * _NC + lax.axis_index("c")
        pltpu.sync_copy(tab.at[w], tab_v)
        svs = [tab_v[pl.ds(16 * i, 16)] for i in range(tab_w // 16)]
        bufs, gsems, ssems = (buf0, buf1), (sg0, sg1), (ss0, ss1)

        def gat(k):
            sv, lane = svs[k // 16], k % 16
            src = lax.squeeze(lax.slice(sv, (lane,), (lane + 1,)), (0,))
            return pltpu.make_async_copy(img.at[src], bufs[k % 2], gsems[k % 2])

        def sca(k):
            return pltpu.make_async_copy(
                bufs[k % 2], out.at[w * bpw + k], ssems[k % 2])

        gat(0).start()
        for k in range(bpw):
            if k + 1 < bpw:
                if k >= 1:
                    sca(k - 1).wait()
                gat(k + 1).start()
            gat(k).wait()
            sca(k).start()
        for j in range(max(0, bpw - 2), bpw):
            sca(j).wait()

    return sc_kernel


_TCK = 8
_TCL = 4


@functools.lru_cache(maxsize=None)
def _make_tc_split(R, Ni, C, H, W):
    def tc_kernel(idx_s, img, out, bufs, gsem, ssem):
        def gat(k):
            slot = lax.rem(k, _TCK)
            return pltpu.make_async_copy(
                img.at[idx_s[k]], bufs.at[slot], gsem.at[slot])

        def sca(k):
            slot = lax.rem(k, _TCK)
            return pltpu.make_async_copy(
                bufs.at[slot], out.at[k], ssem.at[slot])

        for k in range(_TCL):
            gat(k).start()

        def body(i, c):
            kk = i + _TCL

            @pl.when(kk < Ni)
            def _():
                @pl.when(kk >= _TCK)
                def _():
                    sca(kk - _TCK).wait()
                gat(kk).start()

            gat(i).wait()
            sca(i).start()
            return c

        lax.fori_loop(0, Ni, body, 0)

        def drain(j, c):
            sca(j).wait()
            return c

        lax.fori_loop(Ni - _TCK, Ni, drain, 0)

    return pl.pallas_call(
        tc_kernel,
        grid_spec=pltpu.PrefetchScalarGridSpec(
            num_scalar_prefetch=1,
            in_specs=[pl.BlockSpec(memory_space=pl.ANY)],
            out_specs=pl.BlockSpec(memory_space=pl.ANY),
            scratch_shapes=[
                pltpu.VMEM((_TCK, C, H, W), jnp.float32),
                pltpu.SemaphoreType.DMA((_TCK,)),
                pltpu.SemaphoreType.DMA((_TCK,)),
            ],
        ),
        out_shape=jax.ShapeDtypeStruct((Ni, C, H, W), jnp.float32),
    )


def kernel(image, context_indices, target_indices):
    B, V, C, H, W = image.shape
    ni = context_indices.shape[1]
    nt = target_indices.shape[1]
    Ri, Rt = B * ni * C, B * nt * C

    img3 = image.reshape(B * V * C, H, W)
    img4 = image.reshape(B * V, C, H, W)
    bi = jnp.arange(B, dtype=jnp.int32)[:, None]
    ch = jnp.arange(C, dtype=jnp.int32)[None, None, :]
    src_i = (bi * V + context_indices).reshape(-1)
    src_t = (((bi * V + target_indices) * C)[..., None] + ch).reshape(_NW, -1)
    bpw_t = Rt // _NW
    pad = (bpw_t + 15) // 16 * 16 - bpw_t
    tab_t = jnp.concatenate([src_t, jnp.zeros((_NW, pad), jnp.int32)], axis=1)

    out_t = _make_sc_split(B * V * C, Rt, H, W)(img3, tab_t)
    out_i = _make_tc_split(B * V, B * ni, C, H, W)(src_i, img4)
    return (out_i.reshape(B, ni, C, H, W),
            out_t.reshape(B, nt, C, H, W),
            context_indices, target_indices)


# speedup vs baseline: 29.5012x; 1.1751x over previous
import functools

import jax
import jax.numpy as jnp
from jax import lax
from jax.experimental import pallas as pl
from jax.experimental.pallas import tpu as pltpu

_TCK = 8
_TCL = 4


@functools.lru_cache(maxsize=None)
def _make_tc_split(Nv, Ni, Nt, C, H, W):
    n_tot = Ni + Nt

    def tc_kernel(idx_s, img, out_i, out_t, bufs, gsem, ssem):
        def gat(g):
            slot = lax.rem(g, _TCK)
            pltpu.make_async_copy(
                img.at[idx_s[g]], bufs.at[slot], gsem.at[slot]).start()

        def gwait(g):
            slot = lax.rem(g, _TCK)
            pltpu.make_async_copy(img.at[0], bufs.at[slot], gsem.at[slot]).wait()

        def swait(g):
            slot = lax.rem(g, _TCK)
            pltpu.make_async_copy(bufs.at[slot], out_i.at[0], ssem.at[slot]).wait()

        def mk_body(out, base):
            def body(i, c):
                g = base + i
                kk = g + _TCL

                @pl.when(kk < n_tot)
                def _():
                    @pl.when(kk >= _TCK)
                    def _():
                        swait(kk - _TCK)
                    gat(kk)

                gwait(g)
                slot = lax.rem(g, _TCK)
                pltpu.make_async_copy(
                    bufs.at[slot], out.at[i], ssem.at[slot]).start()
                return c
            return body

        for k in range(_TCL):
            gat(k)
        lax.fori_loop(0, Ni, mk_body(out_i, 0), 0)
        lax.fori_loop(0, Nt, mk_body(out_t, Ni), 0)

        def drain(j, c):
            swait(j)
            return c

        lax.fori_loop(n_tot - _TCK, n_tot, drain, 0)

    return pl.pallas_call(
        tc_kernel,
        grid_spec=pltpu.PrefetchScalarGridSpec(
            num_scalar_prefetch=1,
            in_specs=[pl.BlockSpec(memory_space=pl.ANY)],
            out_specs=[pl.BlockSpec(memory_space=pl.ANY),
                       pl.BlockSpec(memory_space=pl.ANY)],
            scratch_shapes=[
                pltpu.VMEM((_TCK, C, H, W), jnp.float32),
                pltpu.SemaphoreType.DMA((_TCK,)),
                pltpu.SemaphoreType.DMA((_TCK,)),
            ],
        ),
        out_shape=[jax.ShapeDtypeStruct((Ni, C, H, W), jnp.float32),
                   jax.ShapeDtypeStruct((Nt, C, H, W), jnp.float32)],
    )


def kernel(image, context_indices, target_indices):
    B, V, C, H, W = image.shape
    ni = context_indices.shape[1]
    nt = target_indices.shape[1]

    img4 = image.reshape(B * V, C, H, W)
    bi = jnp.arange(B, dtype=jnp.int32)[:, None]
    src = jnp.concatenate(
        [(bi * V + context_indices).reshape(-1),
         (bi * V + target_indices).reshape(-1)])

    out_i, out_t = _make_tc_split(B * V, B * ni, B * nt, C, H, W)(src, img4)
    return (out_i.reshape(B, ni, C, H, W),
            out_t.reshape(B, nt, C, H, W),
            context_indices, target_indices)


# speedup vs baseline: 30.8365x; 1.0453x over previous
import functools

import jax
import jax.numpy as jnp
from jax import lax
from jax.experimental import pallas as pl
from jax.experimental.pallas import tpu as pltpu

_TCK = 16
_TCL = 8


@functools.lru_cache(maxsize=None)
def _make_tc_split(Nv, Ni, Nt, C, H, W):
    n_tot = Ni + Nt

    def tc_kernel(idx_s, img, out_i, out_t, bufs, gsem, ssem):
        def gat(g):
            slot = lax.rem(g, _TCK)
            pltpu.make_async_copy(
                img.at[idx_s[g]], bufs.at[slot], gsem.at[slot]).start()

        def gwait(g):
            slot = lax.rem(g, _TCK)
            pltpu.make_async_copy(img.at[0], bufs.at[slot], gsem.at[slot]).wait()

        def swait(g):
            slot = lax.rem(g, _TCK)
            pltpu.make_async_copy(bufs.at[slot], out_i.at[0], ssem.at[slot]).wait()

        def mk_body(out, base):
            def body(i, c):
                g = base + i
                kk = g + _TCL

                @pl.when(kk < n_tot)
                def _():
                    @pl.when(kk >= _TCK)
                    def _():
                        swait(kk - _TCK)
                    gat(kk)

                gwait(g)
                slot = lax.rem(g, _TCK)
                pltpu.make_async_copy(
                    bufs.at[slot], out.at[i], ssem.at[slot]).start()
                return c
            return body

        for k in range(_TCL):
            gat(k)
        lax.fori_loop(0, Ni, mk_body(out_i, 0), 0)
        lax.fori_loop(0, Nt, mk_body(out_t, Ni), 0)

        def drain(j, c):
            swait(j)
            return c

        lax.fori_loop(n_tot - _TCK, n_tot, drain, 0)

    return pl.pallas_call(
        tc_kernel,
        grid_spec=pltpu.PrefetchScalarGridSpec(
            num_scalar_prefetch=1,
            in_specs=[pl.BlockSpec(memory_space=pl.ANY)],
            out_specs=[pl.BlockSpec(memory_space=pl.ANY),
                       pl.BlockSpec(memory_space=pl.ANY)],
            scratch_shapes=[
                pltpu.VMEM((_TCK, C, H, W), jnp.float32),
                pltpu.SemaphoreType.DMA((_TCK,)),
                pltpu.SemaphoreType.DMA((_TCK,)),
            ],
        ),
        out_shape=[jax.ShapeDtypeStruct((Ni, C, H, W), jnp.float32),
                   jax.ShapeDtypeStruct((Nt, C, H, W), jnp.float32)],
    )


def kernel(image, context_indices, target_indices):
    B, V, C, H, W = image.shape
    ni = context_indices.shape[1]
    nt = target_indices.shape[1]

    img4 = image.reshape(B * V, C, H, W)
    bi = jnp.arange(B, dtype=jnp.int32)[:, None]
    src = jnp.concatenate(
        [(bi * V + context_indices).reshape(-1),
         (bi * V + target_indices).reshape(-1)])

    out_i, out_t = _make_tc_split(B * V, B * ni, B * nt, C, H, W)(src, img4)
    return (out_i.reshape(B, ni, C, H, W),
            out_t.reshape(B, nt, C, H, W),
            context_indices, target_indices)


# speedup vs baseline: 30.8459x; 1.0003x over previous
import functools

import jax
import jax.numpy as jnp
from jax import lax
from jax.experimental import pallas as pl
from jax.experimental.pallas import tpu as pltpu

_TCK = 24
_TCL = 12


@functools.lru_cache(maxsize=None)
def _make_tc_split(Nv, Ni, Nt, C, H, W):
    n_tot = Ni + Nt

    def tc_kernel(idx_s, img, out_i, out_t, bufs, gsem, ssem):
        def gat(g):
            slot = lax.rem(g, _TCK)
            pltpu.make_async_copy(
                img.at[idx_s[g]], bufs.at[slot], gsem.at[slot]).start()

        def gwait(g):
            slot = lax.rem(g, _TCK)
            pltpu.make_async_copy(img.at[0], bufs.at[slot], gsem.at[slot]).wait()

        def swait(g):
            slot = lax.rem(g, _TCK)
            pltpu.make_async_copy(bufs.at[slot], out_i.at[0], ssem.at[slot]).wait()

        def mk_body(out, base):
            def body(i, c):
                g = base + i
                kk = g + _TCL

                @pl.when(kk < n_tot)
                def _():
                    @pl.when(kk >= _TCK)
                    def _():
                        swait(kk - _TCK)
                    gat(kk)

                gwait(g)
                slot = lax.rem(g, _TCK)
                pltpu.make_async_copy(
                    bufs.at[slot], out.at[i], ssem.at[slot]).start()
                return c
            return body

        for k in range(_TCL):
            gat(k)
        lax.fori_loop(0, Ni, mk_body(out_i, 0), 0)
        lax.fori_loop(0, Nt, mk_body(out_t, Ni), 0)

        def drain(j, c):
            swait(j)
            return c

        lax.fori_loop(n_tot - _TCK, n_tot, drain, 0)

    return pl.pallas_call(
        tc_kernel,
        grid_spec=pltpu.PrefetchScalarGridSpec(
            num_scalar_prefetch=1,
            in_specs=[pl.BlockSpec(memory_space=pl.ANY)],
            out_specs=[pl.BlockSpec(memory_space=pl.ANY),
                       pl.BlockSpec(memory_space=pl.ANY)],
            scratch_shapes=[
                pltpu.VMEM((_TCK, C, H, W), jnp.float32),
                pltpu.SemaphoreType.DMA((_TCK,)),
                pltpu.SemaphoreType.DMA((_TCK,)),
            ],
        ),
        out_shape=[jax.ShapeDtypeStruct((Ni, C, H, W), jnp.float32),
                   jax.ShapeDtypeStruct((Nt, C, H, W), jnp.float32)],
    )


def kernel(image, context_indices, target_indices):
    B, V, C, H, W = image.shape
    ni = context_indices.shape[1]
    nt = target_indices.shape[1]

    img4 = image.reshape(B * V, C, H, W)
    bi = jnp.arange(B, dtype=jnp.int32)[:, None]
    src = jnp.concatenate(
        [(bi * V + context_indices).reshape(-1),
         (bi * V + target_indices).reshape(-1)])

    out_i, out_t = _make_tc_split(B * V, B * ni, B * nt, C, H, W)(src, img4)
    return (out_i.reshape(B, ni, C, H, W),
            out_t.reshape(B, nt, C, H, W),
            context_indices, target_indices)
